# baseline (speedup 1.0000x reference)
import jax
import jax.numpy as jnp
from jax.experimental import pallas as pl
from jax.experimental.pallas import tpu as pltpu

_K = 8
_C = 1280
_N = 10000


def _body(a_ref, x_ref, wne_ref, bne_ref, wen_ref, ben_ref,
          y_ref, h_ref,
          a_stash, sums_ref, counts_ref, g_ref):
    i = pl.program_id(0)

    @pl.when(i < _K)
    def _phase1():
        col0 = i * _C
        lane = jax.lax.broadcasted_iota(jnp.int32, (1, _C), 1)
        A = jnp.where(col0 + lane < _N, a_ref[:], 0.0)
        row = jax.lax.broadcasted_iota(jnp.int32, (_C, 1), 0)
        X = jnp.where(col0 + row < _N, x_ref[:], 0.0)

        @pl.when(i == 0)
        def _init():
            sums_ref[:] = jnp.zeros_like(sums_ref)
            counts_ref[:] = jnp.zeros_like(counts_ref)

        Ab = A.astype(jnp.bfloat16)
        sums_ref[:] += jax.lax.dot_general(
            Ab, X.astype(jnp.bfloat16), (((1,), (0,)), ((), ())),
            preferred_element_type=jnp.float32)
        counts_ref[:] += jnp.sum(A, axis=1, keepdims=True)
        a_stash[:, pl.ds(col0, _C)] = Ab

    @pl.when(i == _K)
    def _mid():
        mean = sums_ref[:] / jnp.maximum(counts_ref[:], 1.0)
        H = jnp.dot(mean, wne_ref[:],
                    preferred_element_type=jnp.float32) + bne_ref[:]
        h_ref[:] = H
        g_ref[:] = jnp.dot(H, wen_ref[:], preferred_element_type=jnp.float32)

    @pl.when(i >= _K)
    def _phase2():
        Ab = a_stash[:, pl.ds((i - _K) * _C, _C)]
        Gb = g_ref[:].astype(jnp.bfloat16)
        Yagg = jax.lax.dot_general(
            Ab, Gb, (((0,), (0,)), ((), ())),
            preferred_element_type=jnp.float32)
        y_ref[:] = Yagg + ben_ref[:] + x_ref[:]


def kernel(node_features, incidence_matrix, W_ne, b_ne, W_en, b_en):
    n_edges = incidence_matrix.shape[0]
    n_nodes, in_ch = node_features.shape
    out_ch = W_ne.shape[1]
    last = _K - 1

    y, h = pl.pallas_call(
        _body,
        grid=(2 * _K,),
        in_specs=[
            pl.BlockSpec((n_edges, _C), lambda i: (0, jnp.minimum(i, last))),
            pl.BlockSpec((_C, in_ch), lambda i: (i % _K, 0)),
            pl.BlockSpec((in_ch, out_ch), lambda i: (0, 0)),
            pl.BlockSpec((1, out_ch), lambda i: (0, 0)),
            pl.BlockSpec((out_ch, out_ch), lambda i: (0, 0)),
            pl.BlockSpec((1, out_ch), lambda i: (0, 0)),
        ],
        out_specs=(
            pl.BlockSpec((_C, out_ch), lambda i: (i % _K, 0)),
            pl.BlockSpec((n_edges, out_ch), lambda i: (0, 0)),
        ),
        out_shape=(
            jax.ShapeDtypeStruct((n_nodes, out_ch), jnp.float32),
            jax.ShapeDtypeStruct((n_edges, out_ch), jnp.float32),
        ),
        scratch_shapes=[
            pltpu.VMEM((n_edges, _K * _C), jnp.bfloat16),
            pltpu.VMEM((n_edges, out_ch), jnp.float32),
            pltpu.VMEM((n_edges, out_ch), jnp.float32),
            pltpu.VMEM((n_edges, out_ch), jnp.float32),
        ],
    )(incidence_matrix, node_features, W_ne, b_ne.reshape(1, -1),
      W_en, b_en.reshape(1, -1))
    attention_weights = jnp.ones((n_edges,), dtype=jnp.float32)
    return (y, h, attention_weights)


# speedup vs baseline: 1.3053x; 1.3053x over previous
import jax
import jax.numpy as jnp
from jax.experimental import pallas as pl


def _body(a_ref, x_ref, wne_ref, bne_ref, wen_ref, ben_ref, y_ref, h_ref):
    A = a_ref[:]
    Ab = A.astype(jnp.bfloat16)
    Xb = x_ref[:].astype(jnp.bfloat16)

    s1 = jax.lax.dot_general(Ab[:128], Xb, (((1,), (0,)), ((), ())),
                             preferred_element_type=jnp.float32)
    s2 = jax.lax.dot_general(Ab[128:], Xb, (((1,), (0,)), ((), ())),
                             preferred_element_type=jnp.float32)
    sums = jnp.concatenate([s1, s2], axis=0)
    counts = jnp.sum(A, axis=1, keepdims=True)

    mean = sums / jnp.maximum(counts, 1.0)
    H = jnp.dot(mean, wne_ref[:], preferred_element_type=jnp.float32) + bne_ref[:]
    h_ref[:] = H
    G = jnp.dot(H, wen_ref[:], preferred_element_type=jnp.float32)
    Gb = G.astype(jnp.bfloat16)

    y1 = jax.lax.dot_general(Ab[:128], Gb[:128], (((0,), (0,)), ((), ())),
                             preferred_element_type=jnp.float32)
    y2 = jax.lax.dot_general(Ab[128:], Gb[128:], (((0,), (0,)), ((), ())),
                             preferred_element_type=jnp.float32)
    y_ref[:] = (y1 + y2) + ben_ref[:] + x_ref[:]


def kernel(node_features, incidence_matrix, W_ne, b_ne, W_en, b_en):
    n_edges = incidence_matrix.shape[0]
    n_nodes, in_ch = node_features.shape
    out_ch = W_ne.shape[1]
    y, h = pl.pallas_call(
        _body,
        out_shape=(
            jax.ShapeDtypeStruct((n_nodes, out_ch), jnp.float32),
            jax.ShapeDtypeStruct((n_edges, out_ch), jnp.float32),
        ),
    )(incidence_matrix, node_features, W_ne, b_ne.reshape(1, -1),
      W_en, b_en.reshape(1, -1))
    attention_weights = jnp.ones((n_edges,), dtype=jnp.float32)
    return (y, h, attention_weights)
